# baseline (speedup 1.0000x reference)
import functools

import jax
import jax.numpy as jnp
from jax import lax
from jax.experimental import pallas as pl
from jax.experimental.pallas import tpu as pltpu
from jax.experimental.pallas import tpu_sc as plsc

_NC = 2
_NS = 16
_L = 16
_NW = _NC * _NS

_CHUNK = 16384


@functools.lru_cache(maxsize=None)
def _make_bucketize(n: int):
    per_w = n // _NW
    n_chunks = per_w // _CHUNK
    assert per_w % _CHUNK == 0 and n % _NW == 0 and n_chunks % 2 == 0

    mesh = plsc.VectorSubcoreMesh(
        core_axis_name="c", subcore_axis_name="s",
        num_cores=_NC, num_subcores=_NS)

    @functools.partial(
        pl.kernel,
        out_type=jax.ShapeDtypeStruct((n,), jnp.int32),
        mesh=mesh,
        scratch_types=[
            pltpu.VMEM((_CHUNK,), jnp.float32),
            pltpu.VMEM((_CHUNK,), jnp.float32),
            pltpu.VMEM((_CHUNK,), jnp.int32),
            pltpu.VMEM((_CHUNK,), jnp.int32),
            pltpu.SemaphoreType.DMA,
            pltpu.SemaphoreType.DMA,
            pltpu.SemaphoreType.DMA,
            pltpu.SemaphoreType.DMA,
        ],
    )
    def bucketize(x_hbm, out_hbm, xv0, xv1, bv0, bv1, si0, si1, so0, so1):
        wid = lax.axis_index("s") * _NC + lax.axis_index("c")
        base = wid * per_w
        xvs, bvs = (xv0, xv1), (bv0, bv1)
        sin, sout = (si0, si1), (so0, so1)

        def in_copy(k, b):
            return pltpu.make_async_copy(
                x_hbm.at[pl.ds(base + k * _CHUNK, _CHUNK)], xvs[b], sin[b])

        def out_copy(k, b):
            return pltpu.make_async_copy(
                bvs[b], out_hbm.at[pl.ds(base + k * _CHUNK, _CHUNK)], sout[b])

        in_copy(0, 0).start()
        in_copy(1, 1).start()

        def chunk_body(j, carry):
            for b in range(2):
                k = j * 2 + b
                in_copy(k, b).wait()

                @pl.when(k >= 2)
                def _():
                    out_copy(k, b).wait()

                @plsc.parallel_loop(0, _CHUNK // _L, unroll=16)
                def _(i):
                    v = xvs[b][pl.ds(i * _L, _L)]
                    bvs[b][pl.ds(i * _L, _L)] = jnp.where(
                        v >= 0.0,
                        jnp.where(v >= 2.0, jnp.where(v >= 4.0, 3, 2), 1),
                        0)

                out_copy(k, b).start()

                @pl.when(k + 2 < n_chunks)
                def _():
                    in_copy(k + 2, b).start()
            return carry

        lax.fori_loop(0, n_chunks // 2, chunk_body, 0)
        out_copy(n_chunks - 2, 0).wait()
        out_copy(n_chunks - 1, 1).wait()

    return bucketize


def kernel(inputs):
    x = jnp.asarray(inputs, jnp.float32)
    n = x.shape[0]
    ids = _make_bucketize(n)(x.reshape(n)).reshape(n, 1)
    weights = jnp.ones((n, 1), jnp.float32)
    return (ids, weights)


# speedup vs baseline: 1.1928x; 1.1928x over previous
import functools

import jax
import jax.numpy as jnp
from jax import lax
from jax.experimental import pallas as pl
from jax.experimental.pallas import tpu as pltpu
from jax.experimental.pallas import tpu_sc as plsc

_NC = 2
_NS = 16
_L = 16
_NW = _NC * _NS

_CHUNK = 16384


@functools.lru_cache(maxsize=None)
def _make_bucketize(n: int):
    per_w = n // _NW
    n_chunks = per_w // _CHUNK
    assert per_w % _CHUNK == 0 and n % _NW == 0 and n_chunks % 2 == 0

    mesh = plsc.VectorSubcoreMesh(
        core_axis_name="c", subcore_axis_name="s",
        num_cores=_NC, num_subcores=_NS)

    @functools.partial(
        pl.kernel,
        out_type=(jax.ShapeDtypeStruct((n,), jnp.int32),
                  jax.ShapeDtypeStruct((n,), jnp.float32)),
        mesh=mesh,
        scratch_types=[
            pltpu.VMEM((_CHUNK,), jnp.float32),
            pltpu.VMEM((_CHUNK,), jnp.float32),
            pltpu.VMEM((_CHUNK,), jnp.int32),
            pltpu.VMEM((_CHUNK,), jnp.int32),
            pltpu.VMEM((_CHUNK,), jnp.float32),
            pltpu.SemaphoreType.DMA,
            pltpu.SemaphoreType.DMA,
            pltpu.SemaphoreType.DMA,
            pltpu.SemaphoreType.DMA,
            pltpu.SemaphoreType.DMA,
        ],
    )
    def bucketize(x_hbm, out_hbm, w_hbm,
                  xv0, xv1, bv0, bv1, wv, si0, si1, so0, so1, sw):
        wid = lax.axis_index("s") * _NC + lax.axis_index("c")
        base = wid * per_w
        xvs, bvs = (xv0, xv1), (bv0, bv1)
        sin, sout = (si0, si1), (so0, so1)

        def in_copy(k, b):
            return pltpu.make_async_copy(
                x_hbm.at[pl.ds(base + k * _CHUNK, _CHUNK)], xvs[b], sin[b])

        def out_copy(k, b):
            return pltpu.make_async_copy(
                bvs[b], out_hbm.at[pl.ds(base + k * _CHUNK, _CHUNK)], sout[b])

        def w_copy(k):
            return pltpu.make_async_copy(
                wv, w_hbm.at[pl.ds(base + k * _CHUNK, _CHUNK)], sw)

        in_copy(0, 0).start()
        in_copy(1, 1).start()

        @plsc.parallel_loop(0, _CHUNK // _L, unroll=16)
        def _(i):
            wv[pl.ds(i * _L, _L)] = jnp.full((_L,), 1.0, jnp.float32)

        def chunk_body(j, carry):
            for b in range(2):
                k = j * 2 + b
                w_copy(k).start()
                in_copy(k, b).wait()

                @pl.when(k >= 2)
                def _():
                    out_copy(k, b).wait()

                @plsc.parallel_loop(0, _CHUNK // _L, unroll=16)
                def _(i):
                    v = xvs[b][pl.ds(i * _L, _L)]
                    bvs[b][pl.ds(i * _L, _L)] = jnp.where(
                        v >= 0.0,
                        jnp.where(v >= 2.0, jnp.where(v >= 4.0, 3, 2), 1),
                        0)

                out_copy(k, b).start()

                @pl.when(k + 2 < n_chunks)
                def _():
                    in_copy(k + 2, b).start()
            return carry

        lax.fori_loop(0, n_chunks // 2, chunk_body, 0)
        out_copy(n_chunks - 2, 0).wait()
        out_copy(n_chunks - 1, 1).wait()

        def w_drain(j, carry):
            w_copy(0).wait()
            return carry

        lax.fori_loop(0, n_chunks, w_drain, 0)

    return bucketize


def kernel(inputs):
    x = jnp.asarray(inputs, jnp.float32)
    n = x.shape[0]
    ids, weights = _make_bucketize(n)(x.reshape(n))
    return (ids.reshape(n, 1), weights.reshape(n, 1))
